# baseline (speedup 1.0000x reference)
import functools

import jax
import jax.numpy as jnp
from jax.experimental import pallas as pl
from jax.experimental.pallas import tpu as pltpu

_BLOCK_T = 4096


def _router_kernel(x_hbm, wgt_ref, gates_hbm, idx_hbm, aux_ref,
                   xbuf, gbuf, ibuf, gsum_ref, psum_ref,
                   in_sem, g_sem, i_sem, *, block_t, n_tokens):
    i = pl.program_id(0)
    nb = pl.num_programs(0)
    t = block_t
    slot = jax.lax.rem(i, 2)
    nslot = jax.lax.rem(i + 1, 2)
    pslot = nslot

    def in_copy(blk, sl):
        return pltpu.make_async_copy(
            x_hbm.at[pl.ds(blk * t, t), :], xbuf.at[sl], in_sem.at[sl])

    def g_copy(blk, sl):
        return pltpu.make_async_copy(
            gbuf.at[sl], gates_hbm.at[pl.ds(blk * t, t), :], g_sem.at[sl])

    def i_copy(blk, sl):
        return pltpu.make_async_copy(
            ibuf.at[sl], idx_hbm.at[pl.ds(blk * t, t), :], i_sem.at[sl])

    @pl.when(i == 0)
    def _prologue():
        in_copy(0, 0).start()
        gsum_ref[...] = jnp.zeros_like(gsum_ref)
        psum_ref[...] = jnp.zeros_like(psum_ref)

    @pl.when(i + 1 < nb)
    def _prefetch():
        in_copy(i + 1, nslot).start()

    @pl.when(i >= 2)
    def _drain():
        g_copy(i - 2, slot).wait()
        i_copy(i - 2, slot).wait()

    in_copy(i, slot).wait()

    logits = jax.lax.dot_general(
        xbuf[slot], wgt_ref[...], (((1,), (1,)), ((), ())),
        preferred_element_type=jnp.float32)

    m1 = jnp.max(logits, axis=-1, keepdims=True)
    a1 = jnp.argmax(logits, axis=-1)
    eidx = jax.lax.broadcasted_iota(jnp.int32, logits.shape, 1)
    hot1 = eidx == a1[:, None]
    masked = jnp.where(hot1, -jnp.inf, logits)
    m2 = jnp.max(masked, axis=-1, keepdims=True)
    a2 = jnp.argmax(masked, axis=-1)
    hot2 = eidx == a2[:, None]

    tt = jnp.exp(m2 - m1)
    s = 1.0 + tt
    w1 = 1.0 / s
    w2 = tt / s
    gates = jnp.where(hot1, w1, 0.0) + jnp.where(hot2, w2, 0.0)
    gbuf[slot] = gates
    pair = jax.lax.broadcasted_iota(jnp.int32, (a1.shape[0], 2), 1)
    ibuf[slot] = jnp.where(pair == 0, a1[:, None], a2[:, None])

    g_copy(i, slot).start()
    i_copy(i, slot).start()

    p = jnp.exp(logits - m1)
    probs = p / jnp.sum(p, axis=-1, keepdims=True)

    e = gates.shape[1]
    gsum_ref[...] += jnp.sum(gates.reshape(-1, 8, e), axis=0)
    psum_ref[...] += jnp.sum(probs.reshape(-1, 8, e), axis=0)

    @pl.when(i == nb - 1)
    def _epilogue():
        g_copy(i - 1, pslot).wait()
        i_copy(i - 1, pslot).wait()
        g_copy(i, slot).wait()
        i_copy(i, slot).wait()
        scale = jnp.float32(e) / (jnp.float32(n_tokens) ** 2)
        g = jnp.sum(gsum_ref[...], axis=0, keepdims=True)
        q = jnp.sum(psum_ref[...], axis=0, keepdims=True)
        aux_ref[...] = jnp.sum(g * q, keepdims=True) * scale


def kernel(x, w_gate, w_noise):
    del w_noise
    n, d = x.shape
    e = w_gate.shape[0]
    t = _BLOCK_T
    num_blocks = n // t

    gates, idx, aux = pl.pallas_call(
        functools.partial(_router_kernel, block_t=t, n_tokens=n),
        grid=(num_blocks,),
        in_specs=[
            pl.BlockSpec(memory_space=pl.ANY),
            pl.BlockSpec((e, d), lambda i: (0, 0)),
        ],
        out_specs=[
            pl.BlockSpec(memory_space=pl.ANY),
            pl.BlockSpec(memory_space=pl.ANY),
            pl.BlockSpec((1, 1), lambda i: (0, 0)),
        ],
        out_shape=[
            jax.ShapeDtypeStruct((n, e), jnp.float32),
            jax.ShapeDtypeStruct((n, 2), jnp.int32),
            jax.ShapeDtypeStruct((1, 1), jnp.float32),
        ],
        scratch_shapes=[
            pltpu.VMEM((2, t, d), jnp.float32),
            pltpu.VMEM((2, t, e), jnp.float32),
            pltpu.VMEM((2, t, 2), jnp.int32),
            pltpu.VMEM((8, e), jnp.float32),
            pltpu.VMEM((8, e), jnp.float32),
            pltpu.SemaphoreType.DMA((2,)),
            pltpu.SemaphoreType.DMA((2,)),
            pltpu.SemaphoreType.DMA((2,)),
        ],
    )(x, w_gate)
    return gates, idx, aux[0, 0]


# speedup vs baseline: 1.0128x; 1.0128x over previous
import functools

import jax
import jax.numpy as jnp
from jax.experimental import pallas as pl
from jax.experimental.pallas import tpu as pltpu

_BLOCK_T = 4096


def _router_kernel(x_ref, wgt_ref, gates_ref, idx_hbm, aux_ref,
                   ibuf, gsum_ref, psum_ref, i_sem, *, block_t, n_tokens):
    i = pl.program_id(0)
    nb = pl.num_programs(0)
    t = block_t
    slot = jax.lax.rem(i, 2)
    pslot = jax.lax.rem(i + 1, 2)

    def i_copy(blk, sl):
        return pltpu.make_async_copy(
            ibuf.at[sl], idx_hbm.at[pl.ds(blk * t, t), :], i_sem.at[sl])

    @pl.when(i == 0)
    def _init():
        gsum_ref[...] = jnp.zeros_like(gsum_ref)
        psum_ref[...] = jnp.zeros_like(psum_ref)

    @pl.when(i >= 2)
    def _drain():
        i_copy(i - 2, slot).wait()

    logits = jax.lax.dot_general(
        x_ref[...], wgt_ref[...], (((1,), (1,)), ((), ())),
        preferred_element_type=jnp.float32)

    m1 = jnp.max(logits, axis=-1, keepdims=True)
    a1 = jnp.argmax(logits, axis=-1)
    eidx = jax.lax.broadcasted_iota(jnp.int32, logits.shape, 1)
    hot1 = eidx == a1[:, None]
    masked = jnp.where(hot1, -jnp.inf, logits)
    m2 = jnp.max(masked, axis=-1, keepdims=True)
    a2 = jnp.argmax(masked, axis=-1)
    hot2 = eidx == a2[:, None]

    tt = jnp.exp(m2 - m1)
    s = 1.0 + tt
    w1 = 1.0 / s
    w2 = tt / s
    gates = jnp.where(hot1, w1, 0.0) + jnp.where(hot2, w2, 0.0)
    gates_ref[...] = gates
    pair = jax.lax.broadcasted_iota(jnp.int32, (a1.shape[0], 2), 1)
    ibuf[slot] = jnp.where(pair == 0, a1[:, None], a2[:, None])
    i_copy(i, slot).start()

    p = jnp.exp(logits - m1)
    probs = p / jnp.sum(p, axis=-1, keepdims=True)

    e = gates.shape[1]
    gsum_ref[...] += jnp.sum(gates.reshape(-1, 8, e), axis=0)
    psum_ref[...] += jnp.sum(probs.reshape(-1, 8, e), axis=0)

    @pl.when(i == nb - 1)
    def _epilogue():
        i_copy(i - 1, pslot).wait()
        i_copy(i, slot).wait()
        scale = jnp.float32(e) / (jnp.float32(n_tokens) ** 2)
        g = jnp.sum(gsum_ref[...], axis=0, keepdims=True)
        q = jnp.sum(psum_ref[...], axis=0, keepdims=True)
        aux_ref[...] = jnp.sum(g * q, keepdims=True) * scale


def kernel(x, w_gate, w_noise):
    del w_noise
    n, d = x.shape
    e = w_gate.shape[0]
    t = _BLOCK_T
    num_blocks = n // t

    gates, idx, aux = pl.pallas_call(
        functools.partial(_router_kernel, block_t=t, n_tokens=n),
        grid=(num_blocks,),
        in_specs=[
            pl.BlockSpec((t, d), lambda i: (i, 0)),
            pl.BlockSpec((e, d), lambda i: (0, 0)),
        ],
        out_specs=[
            pl.BlockSpec((t, e), lambda i: (i, 0)),
            pl.BlockSpec(memory_space=pl.ANY),
            pl.BlockSpec((1, 1), lambda i: (0, 0)),
        ],
        out_shape=[
            jax.ShapeDtypeStruct((n, e), jnp.float32),
            jax.ShapeDtypeStruct((n, 2), jnp.int32),
            jax.ShapeDtypeStruct((1, 1), jnp.float32),
        ],
        scratch_shapes=[
            pltpu.VMEM((2, t, 2), jnp.int32),
            pltpu.VMEM((8, e), jnp.float32),
            pltpu.VMEM((8, e), jnp.float32),
            pltpu.SemaphoreType.DMA((2,)),
        ],
    )(x, w_gate)
    return gates, idx, aux[0, 0]
